# Initial kernel scaffold; baseline (speedup 1.0000x reference)
import jax
import jax.numpy as jnp
from jax.experimental import pallas as pl


def kernel(boxes, scores):
    raise NotImplementedError("write your pallas kernel here")


# speedup vs baseline: 98.5047x; 98.5047x over previous
import jax
import jax.numpy as jnp
from jax import lax
from jax.experimental import pallas as pl
from jax.experimental.pallas import tpu as pltpu

_NCAND = 2048
_THR = 0.5
_BLK = 256
_NOUT = 100


def _nms_core(boxes_ref, tboxes_ref, data_ref, out_ref, s_scr, l_scr):
    def blk(i, carry):
        bx = boxes_ref[pl.ds(i * _BLK, _BLK), :]
        x1c = bx[:, 0:1]
        y1c = bx[:, 1:2]
        x2c = bx[:, 2:3]
        y2c = bx[:, 3:4]
        area_c = (x2c - x1c) * (y2c - y1c)
        x1r = tboxes_ref[0:1, :]
        y1r = tboxes_ref[1:2, :]
        x2r = tboxes_ref[2:3, :]
        y2r = tboxes_ref[3:4, :]
        area_r = (x2r - x1r) * (y2r - y1r)
        ltx = jnp.maximum(x1c, x1r)
        lty = jnp.maximum(y1c, y1r)
        rbx = jnp.minimum(x2c, x2r)
        rby = jnp.minimum(y2c, y2r)
        w = jnp.maximum(rbx - ltx, 0.0)
        h = jnp.maximum(rby - lty, 0.0)
        inter = w * h
        union = area_c + area_r - inter
        iou = inter / jnp.maximum(union, 1e-9)
        jrow = lax.broadcasted_iota(jnp.int32, (_BLK, _NCAND), 0) + i * _BLK
        icol = lax.broadcasted_iota(jnp.int32, (_BLK, _NCAND), 1)
        s_blk = jnp.where((iou > _THR) & (jrow < icol), 1.0, 0.0)
        l_blk = jnp.where(jrow <= icol, 1.0, 0.0)
        s_scr[pl.ds(i * _BLK, _BLK), :] = s_blk.astype(jnp.bfloat16)
        l_scr[pl.ds(i * _BLK, _BLK), :] = l_blk.astype(jnp.bfloat16)
        return carry
    lax.fori_loop(0, _NCAND // _BLK, blk, 0)

    S = s_scr[...]

    keep0 = jnp.ones((8, _NCAND), jnp.float32)

    def cond(c):
        return c[1]

    def body(c):
        keep, _ = c
        cnt = lax.dot_general(keep.astype(jnp.bfloat16), S,
                              (((1,), (0,)), ((), ())),
                              preferred_element_type=jnp.float32)
        keep_new = jnp.where(cnt == 0.0, 1.0, 0.0)
        ndiff = jnp.sum(jnp.abs(keep_new - keep))
        return keep_new, ndiff > 0.0

    keep, _ = lax.while_loop(cond, body, (keep0, jnp.array(True)))

    Lm = l_scr[...]
    keep_f = keep
    kri = lax.dot_general(keep_f.astype(jnp.bfloat16), Lm,
                          (((1,), (0,)), ((), ())),
                          preferred_element_type=jnp.float32)
    sri = lax.dot_general((1.0 - keep_f).astype(jnp.bfloat16), Lm,
                          (((1,), (0,)), ((), ())),
                          preferred_element_type=jnp.float32)
    nk = jnp.sum(keep_f[0:1, :])
    rank_kept = kri - keep_f
    rank_supp = nk + (sri - (1.0 - keep_f))
    out_rank = jnp.where(keep_f > 0.5, rank_kept, rank_supp)
    rank_row = out_rank[0:1, :]

    rank_i = rank_row.astype(jnp.int32)
    iota_r = lax.broadcasted_iota(jnp.int32, (128, _NCAND), 0)
    oh = (iota_r == jnp.broadcast_to(rank_i, (128, _NCAND))).astype(
        jnp.float32)
    out = lax.dot_general(oh, data_ref[...], (((1,), (0,)), ((), ())),
                          preferred_element_type=jnp.float32,
                          precision=lax.Precision.HIGHEST)
    nk_i = nk.astype(jnp.int32)
    rr = lax.broadcasted_iota(jnp.int32, (128, 8), 0)
    cc = lax.broadcasted_iota(jnp.int32, (128, 8), 1)
    out = jnp.where((rr >= nk_i) & (cc == 4), -jnp.inf, out)
    out_ref[...] = out


def kernel(boxes, scores):
    top_s, top_i = lax.top_k(scores, _NCAND)
    cand = jnp.take(boxes, top_i, axis=0)
    data = jnp.concatenate(
        [cand, top_s[:, None], jnp.zeros((_NCAND, 3), jnp.float32)], axis=1)
    out = pl.pallas_call(
        _nms_core,
        out_shape=jax.ShapeDtypeStruct((128, 8), jnp.float32),
        scratch_shapes=[
            pltpu.VMEM((_NCAND, _NCAND), jnp.bfloat16),
            pltpu.VMEM((_NCAND, _NCAND), jnp.bfloat16),
        ],
    )(cand, cand.T, data)
    return out[:_NOUT, :4], out[:_NOUT, 4]
